# Initial kernel scaffold; baseline (speedup 1.0000x reference)
import jax
import jax.numpy as jnp
from jax.experimental import pallas as pl


def kernel(hidden_states, W1, b1, ln_g, ln_b, W2, b2, k):
    raise NotImplementedError("write your pallas kernel here")


# speedup vs baseline: 1.8581x; 1.8581x over previous
import functools

import jax
import jax.numpy as jnp
from jax import lax
from jax.experimental import pallas as pl
from jax.experimental.pallas import tpu as pltpu

_F = 1536
_H = 512
_TM = 512


def _score_body(x_ref, w1_ref, b1_ref, g_ref, beta_ref, w2a_ref, w2b_ref,
                b2_ref, out_ref):
    xb = x_ref[...].astype(jnp.bfloat16)
    h = lax.dot_general(xb, w1_ref[...].astype(jnp.bfloat16),
                        (((1,), (0,)), ((), ())),
                        preferred_element_type=jnp.float32)
    h = h + b1_ref[...]
    mu = jnp.mean(h, axis=-1, keepdims=True)
    hc = h - mu
    var = jnp.mean(hc * hc, axis=-1, keepdims=True)
    m = jnp.maximum(hc / jnp.sqrt(var + 1e-5) * g_ref[...] + beta_ref[...],
                    0.0)
    s1 = jnp.sum(xb.astype(jnp.float32)
                 * w2a_ref[...].astype(jnp.bfloat16).astype(jnp.float32),
                 axis=-1)
    s2 = jnp.sum(m.astype(jnp.bfloat16).astype(jnp.float32)
                 * w2b_ref[...].astype(jnp.bfloat16).astype(jnp.float32),
                 axis=-1)
    out_ref[0, :] = s1 + s2 + b2_ref[0, 0]


def _mask_body(k_ref, s_ref, out_ref):
    s = s_ref[...]
    kk = k_ref[0]
    mx = jnp.max(s, axis=-1, keepdims=True)
    p = jnp.exp(s - mx)
    lse = mx + jnp.log(jnp.sum(p, axis=-1, keepdims=True))
    u = lax.bitcast_convert_type(s, jnp.uint32)
    key = jnp.where(u >= jnp.uint32(0x80000000), ~u,
                    u | jnp.uint32(0x80000000))

    def body(i, prefix):
        sh = (jnp.int32(31) - i).astype(jnp.uint32)
        cand = prefix | (jnp.uint32(1) << sh)
        cnt = jnp.sum((key >= cand).astype(jnp.int32), axis=-1, keepdims=True)
        return jnp.where(cnt >= kk, cand, prefix)

    thresh = lax.fori_loop(0, 32, body,
                           jnp.zeros((s.shape[0], 1), jnp.uint32))
    out_ref[...] = jnp.where(key >= thresh, s - lse, jnp.float32(-1e9))


def kernel(hidden_states, W1, b1, ln_g, ln_b, W2, b2, k):
    B, N, F = hidden_states.shape
    H = W1.shape[1]
    x2d = hidden_states.reshape(B * N, F)
    G = (B * N) // _TM
    scores = pl.pallas_call(
        _score_body,
        grid=(G,),
        in_specs=[
            pl.BlockSpec((_TM, F), lambda i: (i, 0)),
            pl.BlockSpec((F, H), lambda i: (0, 0)),
            pl.BlockSpec((1, H), lambda i: (0, 0)),
            pl.BlockSpec((1, H), lambda i: (0, 0)),
            pl.BlockSpec((1, H), lambda i: (0, 0)),
            pl.BlockSpec((1, F), lambda i: (0, 0)),
            pl.BlockSpec((1, H), lambda i: (0, 0)),
            pl.BlockSpec((1, 1), lambda i: (0, 0)),
        ],
        out_specs=pl.BlockSpec((1, _TM), lambda i: (0, i)),
        out_shape=jax.ShapeDtypeStruct((1, B * N), jnp.float32),
    )(x2d, W1, b1.reshape(1, H), ln_g.reshape(1, H), ln_b.reshape(1, H),
      W2[:F, 0].reshape(1, F), W2[F:, 0].reshape(1, H), b2.reshape(1, 1))
    scores = scores.reshape(B, N)

    masked = pl.pallas_call(
        _mask_body,
        in_specs=[
            pl.BlockSpec(memory_space=pltpu.SMEM),
            pl.BlockSpec(memory_space=pltpu.VMEM),
        ],
        out_shape=jax.ShapeDtypeStruct((B, N), jnp.float32),
    )(jnp.asarray(k, jnp.int32).reshape(1), scores)
    return masked


# speedup vs baseline: 2.0497x; 1.1031x over previous
import jax
import jax.numpy as jnp
from jax import lax
from jax.experimental import pallas as pl
from jax.experimental.pallas import tpu as pltpu

_TM = 512
_NE = 640


def _score_body(x_ref, w1e_ref, b1_ref, g_ref, beta_ref, w2b_ref, b2_ref,
                out_ref):
    H = 512
    xb = x_ref[...].astype(jnp.bfloat16)
    he = lax.dot_general(xb, w1e_ref[...], (((1,), (0,)), ((), ())),
                         preferred_element_type=jnp.float32)
    h = he[:, :H] + b1_ref[...]
    s1 = he[:, H]
    mu = jnp.mean(h, axis=-1, keepdims=True)
    hc = h - mu
    var = jnp.mean(hc * hc, axis=-1, keepdims=True)
    m = jnp.maximum(hc / jnp.sqrt(var + 1e-5) * g_ref[...] + beta_ref[...],
                    0.0)
    h2 = lax.dot_general(m.astype(jnp.bfloat16), w2b_ref[...],
                         (((1,), (0,)), ((), ())),
                         preferred_element_type=jnp.float32)
    s2 = h2[:, 0]
    out_ref[0, :] = s1 + s2 + b2_ref[0, 0]


def _mask_body(k_ref, s_ref, out_ref):
    s = s_ref[...]
    kk = k_ref[0]
    mx = jnp.max(s, axis=-1, keepdims=True)
    p = jnp.exp(s - mx)
    lse = mx + jnp.log(jnp.sum(p, axis=-1, keepdims=True))
    u = lax.bitcast_convert_type(s, jnp.uint32)
    key = jnp.where(u >= jnp.uint32(0x80000000), ~u,
                    u | jnp.uint32(0x80000000))

    def body(i, prefix):
        sh = (jnp.int32(31) - i).astype(jnp.uint32)
        cand = prefix | (jnp.uint32(1) << sh)
        cnt = jnp.sum((key >= cand).astype(jnp.int32), axis=-1, keepdims=True)
        return jnp.where(cnt >= kk, cand, prefix)

    thresh = lax.fori_loop(0, 32, body,
                           jnp.zeros((s.shape[0], 1), jnp.uint32))
    out_ref[...] = jnp.where(key >= thresh, s - lse, jnp.float32(-1e9))


def kernel(hidden_states, W1, b1, ln_g, ln_b, W2, b2, k):
    B, N, F = hidden_states.shape
    H = W1.shape[1]
    x2d = hidden_states.reshape(B * N, F)
    w1e = jnp.concatenate(
        [W1, W2[:F], jnp.zeros((F, _NE - H - 1), W1.dtype)],
        axis=1).astype(jnp.bfloat16)
    w2bp = jnp.concatenate(
        [W2[F:], jnp.zeros((H, 127), W2.dtype)], axis=1).astype(jnp.bfloat16)
    G = (B * N) // _TM
    scores = pl.pallas_call(
        _score_body,
        grid=(G,),
        in_specs=[
            pl.BlockSpec((_TM, F), lambda i: (i, 0)),
            pl.BlockSpec((F, _NE), lambda i: (0, 0)),
            pl.BlockSpec((1, H), lambda i: (0, 0)),
            pl.BlockSpec((1, H), lambda i: (0, 0)),
            pl.BlockSpec((1, H), lambda i: (0, 0)),
            pl.BlockSpec((H, 128), lambda i: (0, 0)),
            pl.BlockSpec((1, 1), lambda i: (0, 0)),
        ],
        out_specs=pl.BlockSpec((1, _TM), lambda i: (0, i)),
        out_shape=jax.ShapeDtypeStruct((1, B * N), jnp.float32),
    )(x2d, w1e, b1.reshape(1, H), ln_g.reshape(1, H), ln_b.reshape(1, H),
      w2bp, b2.reshape(1, 1))
    scores = scores.reshape(B, N)

    masked = pl.pallas_call(
        _mask_body,
        in_specs=[
            pl.BlockSpec(memory_space=pltpu.SMEM),
            pl.BlockSpec(memory_space=pltpu.VMEM),
        ],
        out_shape=jax.ShapeDtypeStruct((B, N), jnp.float32),
    )(jnp.asarray(k, jnp.int32).reshape(1), scores)
    return masked
